# baseline (speedup 1.0000x reference)
import functools

import jax
import jax.numpy as jnp
from jax import lax
from jax.experimental import pallas as pl
from jax.experimental.pallas import tpu as pltpu
from jax.experimental.pallas import tpu_sc as plsc

VOCAB = 1_000_000
EMBED = 64
BATCH = 16384
LANES = 16

_info = plsc.get_sparse_core_info()
NC = _info.num_cores
NS = _info.num_subcores
NW = NC * NS
BLK = 512
BLKSH = 9
NBF = VOCAB // BLK
BPW = NBF // NW
NEXTRA = NBF - BPW * NW
TAILS = NBF * BLK
TAILW = VOCAB - TAILS
IDXCH = 1024
NBKT = 64
BCAP = 48
RING = 128
NV = EMBED // LANES


def _take16(x, idx):
    return lax.gather(
        x,
        idx[:, None],
        dimension_numbers=lax.GatherDimensionNumbers(
            offset_dims=(), collapsed_slice_dims=(0,), start_index_map=(0,)
        ),
        slice_sizes=(1,),
        mode=lax.GatherScatterMode.PROMISE_IN_BOUNDS,
    )


def _rsqrt(x):
    i = lax.bitcast_convert_type(x, jnp.int32)
    i = jnp.int32(0x5F3759DF) - (i >> 1)
    y = lax.bitcast_convert_type(i, jnp.float32)
    for _ in range(3):
        y = y * (1.5 - 0.5 * x * y * y)
    return y


@functools.partial(
    pl.kernel,
    mesh=plsc.VectorSubcoreMesh(core_axis_name="c", subcore_axis_name="s"),
    out_type=jax.ShapeDtypeStruct((BATCH * EMBED,), jnp.float32),
    scratch_types=[
        pltpu.VMEM((IDXCH,), jnp.int32),
        pltpu.VMEM((2, EMBED, BLK), jnp.float32),
        pltpu.VMEM((EMBED, TAILW), jnp.float32),
        pltpu.VMEM((NBKT,), jnp.int32),
        pltpu.VMEM((NBKT * BCAP,), jnp.int32),
        pltpu.VMEM((NBKT * BCAP,), jnp.int32),
        pltpu.VMEM((RING, EMBED), jnp.float32),
        pltpu.SemaphoreType.DMA,
        pltpu.SemaphoreType.DMA,
        pltpu.SemaphoreType.DMA,
    ],
    compiler_params=pltpu.CompilerParams(needs_layout_passes=False),
)
def _embed_norm(
    table_hbm, idx_hbm, out_hbm, idx_v, blk_v, tail_v, bkt_n, bkt_i, bkt_b,
    colbuf, sem_out, sem_b0, sem_b1,
):
    wid = lax.axis_index("s") * NC + lax.axis_index("c")
    nb = BPW + (wid < NEXTRA).astype(jnp.int32)
    lo = BLK * (BPW * wid + jnp.minimum(wid, NEXTRA))
    hi = jnp.where(wid == NW - 1, jnp.int32(VOCAB), lo + nb * BLK)
    lanes = lax.iota(jnp.int32, LANES)
    lane0 = lanes == 0
    zeros16 = jnp.zeros((LANES,), jnp.int32)

    def _zero(v, carry):
        bkt_n[pl.ds(v * LANES, LANES)] = zeros16
        return carry

    lax.fori_loop(0, NBKT // LANES, _zero, 0)

    def _chunk(ch, carry):
        pltpu.sync_copy(idx_hbm.at[pl.ds(ch * IDXCH, IDXCH)], idx_v)

        def _sel(v, carry):
            x = idx_v[pl.ds(v * LANES, LANES)]
            b = lanes + (ch * IDXCH + v * LANES)
            m = (x >= lo) & (x < hi)
            n = plsc.all_reduce_population_count(m)[0]

            def _body(cr):
                mi, k = cr
                l = plsc.all_reduce_ffs(mi != 0)[0]
                lsp = jnp.full((LANES,), l, dtype=jnp.int32)
                xi = _take16(x, lsp)
                bi = _take16(b, lsp)
                bid = (xi - lo) >> BLKSH
                cvec = bkt_n[pl.ds((bid[0] >> 4) * LANES, LANES)]
                c = _take16(cvec, bid & (LANES - 1))
                cc = jnp.minimum(c, BCAP - 1)
                slot = bid * BCAP + cc
                plsc.store_scatter(bkt_i, [slot], xi, mask=lane0)
                plsc.store_scatter(bkt_b, [slot], bi, mask=lane0)
                plsc.store_scatter(bkt_n, [bid], c + 1, mask=lane0)
                return (jnp.where(lanes == l, jnp.int32(0), mi), k - 1)

            lax.while_loop(
                lambda cr: cr[1] > 0, _body, (m.astype(jnp.int32), n)
            )
            return carry

        return lax.fori_loop(0, IDXCH // LANES, _sel, carry)

    lax.fori_loop(0, BATCH // IDXCH, _chunk, 0)

    def _drain1(c, carry):
        pltpu.make_async_copy(
            out_hbm.at[pl.ds(0, EMBED)], colbuf.at[0], sem_out
        ).wait()
        return carry

    def _run(buf, ci_all, eb, cnt, st):
        def _go(st):
            sc, dr = st
            mvalid = lanes < cnt
            civ = jnp.where(mvalid, ci_all, 0)
            need = sc + cnt - dr - (RING - 16)
            dr = dr + lax.fori_loop(
                0, jnp.maximum(need, 0), lambda c, n: _drain1(c, n) + 1, 0
            )
            acc = jnp.zeros((LANES,), jnp.float32)
            for d in range(EMBED):
                dv = jnp.full((LANES,), d, dtype=jnp.int32)
                v = plsc.load_gather(buf, [dv, civ])
                acc = acc + v * v
            y = _rsqrt(acc + 1e-12)
            slots = (sc + lanes) & (RING - 1)
            for d in range(EMBED):
                dv = jnp.full((LANES,), d, dtype=jnp.int32)
                v = plsc.load_gather(buf, [dv, civ])
                plsc.store_scatter(colbuf, [slots, dv], v * y, mask=mvalid)

            def _emit(j, carry):
                jsp = jnp.full((LANES,), j, dtype=jnp.int32)
                bb = _take16(eb, jsp)[0]
                pltpu.async_copy(
                    colbuf.at[(sc + j) & (RING - 1)],
                    out_hbm.at[pl.ds(bb * EMBED, EMBED)],
                    sem_out,
                )
                return carry

            lax.fori_loop(0, cnt, _emit, 0)
            return (sc + cnt, dr)

        return lax.cond(cnt > 0, _go, lambda st: st, st)

    def _match(buf, start, bid, st):
        cvec = bkt_n[pl.ds((bid >> 4) * LANES, LANES)]
        c = _take16(cvec, jnp.full((LANES,), bid & (LANES - 1), jnp.int32))[0]
        c = jnp.minimum(c, BCAP)
        for half in range(BCAP // LANES):
            ei = bkt_i[pl.ds(bid * BCAP + half * LANES, LANES)]
            eb = bkt_b[pl.ds(bid * BCAP + half * LANES, LANES)]
            cnt = jnp.clip(c - half * LANES, 0, LANES)
            st = _run(buf, ei - start, eb, cnt, st)
        return st

    sems = (sem_b0, sem_b1)

    def _start(kk, slot):
        def _f(c):
            h = BLK // 2
            pltpu.async_copy(
                table_hbm.at[:, pl.ds(lo + kk * BLK, h)],
                blk_v.at[slot].at[:, pl.ds(0, h)],
                sems[slot],
            )
            pltpu.async_copy(
                table_hbm.at[:, pl.ds(lo + kk * BLK + h, h)],
                blk_v.at[slot].at[:, pl.ds(h, h)],
                sems[slot],
            )
            return c

        lax.cond(kk < nb, _f, lambda c: c, 0)

    def _consume(kk, slot, st):
        def _f(st):
            pltpu.make_async_copy(
                table_hbm.at[:, pl.ds(0, BLK)], blk_v.at[slot], sems[slot]
            ).wait()
            return _match(blk_v.at[slot], lo + kk * BLK, kk, st)

        return lax.cond(kk < nb, _f, lambda st: st, st)

    _start(jnp.int32(0), 0)

    def _pair(g, st):
        _start(2 * g + 1, 1)
        st = _consume(2 * g, 0, st)
        _start(2 * g + 2, 0)
        st = _consume(2 * g + 1, 1, st)
        return st

    st = lax.fori_loop(
        0, (BPW + 1) // 2 + 1, _pair, (jnp.int32(0), jnp.int32(0))
    )

    def _tail(st):
        pltpu.sync_copy(table_hbm.at[:, pl.ds(TAILS, TAILW)], tail_v)
        return _match(tail_v, jnp.int32(TAILS), nb, st)

    st = lax.cond(wid == NW - 1, _tail, lambda st: st, st)

    lax.fori_loop(0, st[0] - st[1], _drain1, 0)


def kernel(indices, table):
    idx = indices.astype(jnp.int32)
    res = _embed_norm(table.T, idx)
    return res.reshape(BATCH, EMBED)


# speedup vs baseline: 1.0260x; 1.0260x over previous
import functools

import jax
import jax.numpy as jnp
from jax import lax
from jax.experimental import pallas as pl
from jax.experimental.pallas import tpu as pltpu
from jax.experimental.pallas import tpu_sc as plsc

VOCAB = 1_000_000
EMBED = 64
BATCH = 16384
LANES = 16

_info = plsc.get_sparse_core_info()
NC = _info.num_cores
NS = _info.num_subcores
NW = NC * NS
BLK = 512
BLKSH = 9
NBF = VOCAB // BLK
BPW = NBF // NW
NEXTRA = NBF - BPW * NW
TAILS = NBF * BLK
TAILW = VOCAB - TAILS
IDXCH = 1024
NBKT = 64
BCAP = 48
RING = 128
NV = EMBED // LANES


def _take16(x, idx):
    return lax.gather(
        x,
        idx[:, None],
        dimension_numbers=lax.GatherDimensionNumbers(
            offset_dims=(), collapsed_slice_dims=(0,), start_index_map=(0,)
        ),
        slice_sizes=(1,),
        mode=lax.GatherScatterMode.PROMISE_IN_BOUNDS,
    )


def _rsqrt(x):
    i = lax.bitcast_convert_type(x, jnp.int32)
    i = jnp.int32(0x5F3759DF) - (i >> 1)
    y = lax.bitcast_convert_type(i, jnp.float32)
    for _ in range(3):
        y = y * (1.5 - 0.5 * x * y * y)
    return y


@functools.partial(
    pl.kernel,
    mesh=plsc.VectorSubcoreMesh(core_axis_name="c", subcore_axis_name="s"),
    out_type=jax.ShapeDtypeStruct((BATCH * EMBED,), jnp.float32),
    scratch_types=[
        pltpu.VMEM((IDXCH,), jnp.int32),
        pltpu.VMEM((2, EMBED, BLK), jnp.float32),
        pltpu.VMEM((EMBED, TAILW), jnp.float32),
        pltpu.VMEM((NBKT,), jnp.int32),
        pltpu.VMEM((NBKT * BCAP,), jnp.int32),
        pltpu.VMEM((NBKT * BCAP,), jnp.int32),
        pltpu.VMEM((RING, EMBED), jnp.float32),
        pltpu.SemaphoreType.DMA,
        pltpu.SemaphoreType.DMA,
        pltpu.SemaphoreType.DMA,
    ],
    compiler_params=pltpu.CompilerParams(needs_layout_passes=False),
)
def _embed_norm(
    table_hbm, idx_hbm, out_hbm, idx_v, blk_v, tail_v, bkt_n, bkt_i, bkt_b,
    colbuf, sem_out, sem_b0, sem_b1,
):
    wid = lax.axis_index("s") * NC + lax.axis_index("c")
    nb = BPW + (wid < NEXTRA).astype(jnp.int32)
    lo = BLK * (BPW * wid + jnp.minimum(wid, NEXTRA))
    hi = jnp.where(wid == NW - 1, jnp.int32(VOCAB), lo + nb * BLK)
    lanes = lax.iota(jnp.int32, LANES)
    lane0 = lanes == 0
    zeros16 = jnp.zeros((LANES,), jnp.int32)

    def _zero(v, carry):
        bkt_n[pl.ds(v * LANES, LANES)] = zeros16
        return carry

    lax.fori_loop(0, NBKT // LANES, _zero, 0)

    def _chunk(ch, carry):
        pltpu.sync_copy(idx_hbm.at[pl.ds(ch * IDXCH, IDXCH)], idx_v)

        def _sel(v, carry):
            x = idx_v[pl.ds(v * LANES, LANES)]
            b = lanes + (ch * IDXCH + v * LANES)
            m = (x >= lo) & (x < hi)
            n = plsc.all_reduce_population_count(m)[0]

            def _body(cr):
                mi, k = cr
                l = plsc.all_reduce_ffs(mi != 0)[0]
                lsp = jnp.full((LANES,), l, dtype=jnp.int32)
                xi = _take16(x, lsp)
                bi = _take16(b, lsp)
                bid = (xi - lo) >> BLKSH
                cvec = bkt_n[pl.ds((bid[0] >> 4) * LANES, LANES)]
                c = _take16(cvec, bid & (LANES - 1))
                cc = jnp.minimum(c, BCAP - 1)
                slot = bid * BCAP + cc
                plsc.store_scatter(bkt_i, [slot], xi, mask=lane0)
                plsc.store_scatter(bkt_b, [slot], bi, mask=lane0)
                plsc.store_scatter(bkt_n, [bid], c + 1, mask=lane0)
                return (jnp.where(lanes == l, jnp.int32(0), mi), k - 1)

            lax.while_loop(
                lambda cr: cr[1] > 0, _body, (m.astype(jnp.int32), n)
            )
            return carry

        return lax.fori_loop(0, IDXCH // LANES, _sel, carry)

    lax.fori_loop(0, BATCH // IDXCH, _chunk, 0)

    def _drain1(c, carry):
        pltpu.make_async_copy(
            out_hbm.at[pl.ds(0, EMBED)], colbuf.at[0], sem_out
        ).wait()
        return carry

    def _run(buf, ci_all, eb, cnt, st):
        def _go(st):
            sc, dr = st
            mvalid = lanes < cnt
            civ = jnp.where(mvalid, ci_all, 0)
            need = sc + cnt - dr - (RING - 16)
            dr = dr + lax.fori_loop(
                0, jnp.maximum(need, 0), lambda c, n: _drain1(c, n) + 1, 0
            )
            acc = jnp.zeros((LANES,), jnp.float32)
            for d in range(EMBED):
                dv = jnp.full((LANES,), d, dtype=jnp.int32)
                v = plsc.load_gather(buf, [dv, civ])
                acc = acc + v * v
            y = _rsqrt(acc + 1e-12)
            slots = (sc + lanes) & (RING - 1)
            for d in range(EMBED):
                dv = jnp.full((LANES,), d, dtype=jnp.int32)
                v = plsc.load_gather(buf, [dv, civ])
                plsc.store_scatter(colbuf, [slots, dv], v * y, mask=mvalid)

            def _emit(j, carry):
                jsp = jnp.full((LANES,), j, dtype=jnp.int32)
                bb = _take16(eb, jsp)[0]
                pltpu.async_copy(
                    colbuf.at[(sc + j) & (RING - 1)],
                    out_hbm.at[pl.ds(bb * EMBED, EMBED)],
                    sem_out,
                )
                return carry

            lax.fori_loop(0, cnt, _emit, 0)
            return (sc + cnt, dr)

        return lax.cond(cnt > 0, _go, lambda st: st, st)

    def _match(buf, start, bid, st):
        cvec = bkt_n[pl.ds((bid >> 4) * LANES, LANES)]
        c = _take16(cvec, jnp.full((LANES,), bid & (LANES - 1), jnp.int32))[0]
        c = jnp.minimum(c, BCAP)
        for half in range(BCAP // LANES):
            ei = bkt_i[pl.ds(bid * BCAP + half * LANES, LANES)]
            eb = bkt_b[pl.ds(bid * BCAP + half * LANES, LANES)]
            cnt = jnp.clip(c - half * LANES, 0, LANES)
            st = _run(buf, ei - start, eb, cnt, st)
        return st

    sems = (sem_b0, sem_b1)

    def _start(kk, slot):
        def _f(c):
            for r in range(EMBED // 8):
                pltpu.async_copy(
                    table_hbm.at[pl.ds(8 * r, 8), pl.ds(lo + kk * BLK, BLK)],
                    blk_v.at[slot].at[pl.ds(8 * r, 8)],
                    sems[slot],
                )
            return c

        lax.cond(kk < nb, _f, lambda c: c, 0)

    def _consume(kk, slot, st):
        def _f(st):
            pltpu.make_async_copy(
                table_hbm.at[:, pl.ds(0, BLK)], blk_v.at[slot], sems[slot]
            ).wait()
            return _match(blk_v.at[slot], lo + kk * BLK, kk, st)

        return lax.cond(kk < nb, _f, lambda st: st, st)

    _start(jnp.int32(0), 0)

    def _pair(g, st):
        _start(2 * g + 1, 1)
        st = _consume(2 * g, 0, st)
        _start(2 * g + 2, 0)
        st = _consume(2 * g + 1, 1, st)
        return st

    st = lax.fori_loop(
        0, (BPW + 1) // 2 + 1, _pair, (jnp.int32(0), jnp.int32(0))
    )

    def _tail(st):
        pltpu.sync_copy(table_hbm.at[:, pl.ds(TAILS, TAILW)], tail_v)
        return _match(tail_v, jnp.int32(TAILS), nb, st)

    st = lax.cond(wid == NW - 1, _tail, lambda st: st, st)

    lax.fori_loop(0, st[0] - st[1], _drain1, 0)


def kernel(indices, table):
    idx = indices.astype(jnp.int32)
    res = _embed_norm(table.T, idx)
    return res.reshape(BATCH, EMBED)
